# baseline (speedup 1.0000x reference)
import functools
import math

import jax
import jax.numpy as jnp
import numpy as np
from jax import lax
from jax.experimental import pallas as pl
from jax.experimental.pallas import tpu as pltpu
from jax.experimental.pallas import tpu_sc as plsc

D = 128
SEQ = 200
SCALE = math.sqrt(float(D))


def _positional_encoding(length, depth):
    half = depth // 2
    positions = np.arange(length)[:, None].astype(np.float32)
    depths = np.arange(half)[None, :].astype(np.float32) / float(half)
    angle_rates = 1.0 / (10000.0 ** depths)
    angle_rads = positions * angle_rates
    return np.concatenate([np.sin(angle_rads), np.cos(angle_rads)], axis=-1)


_PE = jnp.asarray(_positional_encoding(2048, D)[:SEQ], dtype=jnp.float32)


@functools.cache
def _make_kernel(n_batch):
    info = plsc.get_sparse_core_info()
    nc, ns = info.num_cores, info.num_subcores
    nw = nc * ns
    seqs_per_w = n_batch // nw
    mesh = plsc.VectorSubcoreMesh(core_axis_name="c", subcore_axis_name="s")

    @functools.partial(
        pl.kernel,
        out_type=jax.ShapeDtypeStruct((n_batch, SEQ, D), jnp.float32),
        mesh=mesh,
        scratch_types=[
            pltpu.VMEM((seqs_per_w * SEQ,), jnp.int32),
            pltpu.VMEM((SEQ, D), jnp.float32),
            pltpu.VMEM((SEQ, D), jnp.float32),
            pltpu.VMEM((SEQ, D), jnp.float32),
            pltpu.SemaphoreType.DMA,
            pltpu.SemaphoreType.DMA,
            pltpu.SemaphoreType.DMA,
            pltpu.SemaphoreType.DMA,
        ],
    )
    def k(x_hbm, table_hbm, pe_hbm, out_hbm, idx_v, pe_v, rows0, rows1,
          g0, g1, o0, o1):
        wid = lax.axis_index("s") * nc + lax.axis_index("c")
        base = wid * seqs_per_w * SEQ
        pltpu.sync_copy(x_hbm.at[pl.ds(base, seqs_per_w * SEQ)], idx_v)
        pltpu.sync_copy(pe_hbm, pe_v)

        rows = (rows0, rows1)
        gsem = (g0, g1)
        osem = (o0, o1)

        def gather_src(i):
            return table_hbm.at[idx_v.at[pl.ds(i * SEQ, SEQ)]]

        def out_dst(i):
            return out_hbm.at[wid * seqs_per_w + i]

        def compute(b):
            buf = rows[b]

            def row_body(t, c2):
                for g in range(D // 16):
                    sl = pl.ds(g * 16, 16)
                    buf[t, sl] = buf[t, sl] * SCALE + pe_v[t, sl]
                return c2

            lax.fori_loop(0, SEQ, row_body, 0, unroll=2)

        pltpu.async_copy(gather_src(0), rows[0], gsem[0])
        for i in range(seqs_per_w):
            b = i & 1
            pltpu.make_async_copy(gather_src(i), rows[b], gsem[b]).wait()
            if i + 1 < seqs_per_w:
                if i >= 1:
                    pltpu.make_async_copy(
                        rows[1 - b], out_dst(i - 1), osem[1 - b]).wait()
                pltpu.async_copy(gather_src(i + 1), rows[1 - b], gsem[1 - b])
            compute(b)
            pltpu.async_copy(rows[b], out_dst(i), osem[b])
        pltpu.make_async_copy(
            rows[0], out_dst(seqs_per_w - 2), osem[0]).wait()
        pltpu.make_async_copy(
            rows[1], out_dst(seqs_per_w - 1), osem[1]).wait()

    return k


def kernel(x, table):
    n_batch = x.shape[0]
    return _make_kernel(n_batch)(x.reshape(-1), table, _PE)


# speedup vs baseline: 1.1320x; 1.1320x over previous
import functools
import math

import jax
import jax.numpy as jnp
import numpy as np
from jax import lax
from jax.experimental import pallas as pl
from jax.experimental.pallas import tpu as pltpu
from jax.experimental.pallas import tpu_sc as plsc

D = 128
SEQ = 200
NBUF = 3
SCALE = math.sqrt(float(D))


def _positional_encoding(length, depth):
    half = depth // 2
    positions = np.arange(length)[:, None].astype(np.float32)
    depths = np.arange(half)[None, :].astype(np.float32) / float(half)
    angle_rates = 1.0 / (10000.0 ** depths)
    angle_rads = positions * angle_rates
    return np.concatenate([np.sin(angle_rads), np.cos(angle_rads)], axis=-1)


_PE = jnp.asarray(_positional_encoding(2048, D)[:SEQ], dtype=jnp.float32)


@functools.cache
def _make_kernel(n_batch):
    info = plsc.get_sparse_core_info()
    nc, ns = info.num_cores, info.num_subcores
    nw = nc * ns
    spw = n_batch // nw
    mesh = plsc.VectorSubcoreMesh(core_axis_name="c", subcore_axis_name="s")
    n_groups = (spw - 2) // NBUF

    @functools.partial(
        pl.kernel,
        out_type=jax.ShapeDtypeStruct((n_batch * SEQ, D), jnp.float32),
        mesh=mesh,
        scratch_types=[
            pltpu.VMEM((spw * SEQ,), jnp.int32),
            pltpu.VMEM((SEQ, D), jnp.float32),
        ] + [pltpu.VMEM((SEQ, D), jnp.float32)] * NBUF
          + [pltpu.SemaphoreType.DMA] * (2 * NBUF),
    )
    def k(x_hbm, table_hbm, pe_hbm, out_hbm, idx_v, pe_v,
          r0, r1, r2, g0, g1, g2, o0, o1, o2):
        rows = (r0, r1, r2)
        gsem = (g0, g1, g2)
        osem = (o0, o1, o2)
        wid = lax.axis_index("s") * nc + lax.axis_index("c")
        pltpu.sync_copy(x_hbm.at[pl.ds(wid * spw * SEQ, spw * SEQ)], idx_v)
        pltpu.sync_copy(pe_hbm, pe_v)
        row_base = wid * spw * SEQ

        def gather(u, b):
            return pltpu.make_async_copy(
                table_hbm.at[idx_v.at[pl.ds(u * SEQ, SEQ)]], rows[b],
                gsem[b])

        def out_cp(u, b):
            return pltpu.make_async_copy(
                rows[b], out_hbm.at[pl.ds(row_base + u * SEQ, SEQ)],
                osem[b])

        def compute(b):
            buf = rows[b]

            def row_body(t, c):
                for g in range(D // 16):
                    sl = pl.ds(g * 16, 16)
                    buf[t, sl] = buf[t, sl] * SCALE + pe_v[t, sl]
                return c

            lax.fori_loop(0, SEQ, row_body, 0, unroll=2)

        gather(0, 0).start()
        gather(1, 1).start()

        def group(p, carry):
            for j in range(NBUF):
                u = NBUF * p + j
                nb = (j + 2) % NBUF
                gather(u, j).wait()
                compute(j)
                out_cp(u, j).start()

                @pl.when(u >= 1)
                def _():
                    out_cp(u - 1, nb).wait()

                gather(u + 2, nb).start()
            return carry

        lax.fori_loop(0, n_groups, group, 0)

        for u, j in ((spw - 2, 0), (spw - 1, 1)):
            gather(u, j).wait()
            compute(j)
            out_cp(u, j).start()
        out_cp(spw - 3, 2).wait()
        out_cp(spw - 2, 0).wait()
        out_cp(spw - 1, 1).wait()

    return k


def kernel(x, table):
    n_batch, seq = x.shape
    out = _make_kernel(n_batch)(x.reshape(-1), table, _PE)
    return out.reshape(n_batch, seq, D)
